# Initial kernel scaffold; baseline (speedup 1.0000x reference)
import functools

import jax
import jax.numpy as jnp
from jax import lax
from jax.experimental import pallas as pl
from jax.experimental.pallas import tpu as pltpu
from jax.experimental.pallas import tpu_sc as plsc

N = 10000
E = 320000
D = 128

NC = 2
NS = 16
NW = NC * NS
C = 128
CPW = 79
E_PAD = NW * CPW * C
N_PAD = 10016
RPT = N_PAD // NS


def _sc_segsum(with_cnt):
  mesh = plsc.VectorSubcoreMesh(core_axis_name="c", subcore_axis_name="s")
  out_type = [jax.ShapeDtypeStruct((NC, N_PAD, D), jnp.float32)]
  if with_cnt:
    out_type.append(jax.ShapeDtypeStruct((NC, N_PAD, 16), jnp.float32))
  scratch_types = [
      pltpu.VMEM((CPW, C), jnp.int32),
      pltpu.VMEM((CPW, C), jnp.int32),
      pltpu.VMEM((C, D), jnp.float32),
      pltpu.VMEM_SHARED((N_PAD, D), jnp.float32),
  ]
  if with_cnt:
    scratch_types += [
        pltpu.VMEM((C, 16), jnp.float32),
        pltpu.VMEM((RPT, 16), jnp.float32),
        pltpu.VMEM_SHARED((N_PAD, 16), jnp.float32),
    ]

  def body(table_hbm, src_hbm, dst_hbm, *refs):
    if with_cnt:
      (out_sum, out_cnt, src_v, dst_v, rows_v, psum_sp,
       ones_v, z16_v, cnt_sp) = refs
    else:
      out_sum, src_v, dst_v, rows_v, psum_sp = refs
    cid = lax.axis_index("c")
    sid = lax.axis_index("s")
    wid = cid * NS + sid

    def zrow(i, carry):
      r = i // (D // 16)
      g = i % (D // 16)
      rows_v[r, pl.ds(g * 16, 16)] = jnp.zeros((16,), jnp.float32)
      return carry
    lax.fori_loop(0, C * (D // 16), zrow, 0)
    base = sid * RPT
    for k in range(RPT // C):
      pltpu.sync_copy(rows_v, psum_sp.at[pl.ds(base + k * C, C)])
    rem = RPT % C
    if rem:
      pltpu.sync_copy(rows_v.at[pl.ds(0, rem)],
                      psum_sp.at[pl.ds(base + (RPT // C) * C, rem)])

    if with_cnt:
      def orow(i, carry):
        ones_v[i, pl.ds(0, 16)] = jnp.ones((16,), jnp.float32)
        return carry
      lax.fori_loop(0, C, orow, 0)
      def z16row(i, carry):
        z16_v[i, pl.ds(0, 16)] = jnp.zeros((16,), jnp.float32)
        return carry
      lax.fori_loop(0, RPT, z16row, 0)
      pltpu.sync_copy(z16_v, cnt_sp.at[pl.ds(base, RPT)])

    pltpu.sync_copy(src_hbm.at[wid], src_v)
    pltpu.sync_copy(dst_hbm.at[wid], dst_v)

    plsc.subcore_barrier()

    def step(j, carry):
      pltpu.sync_copy(table_hbm.at[src_v.at[j]], rows_v)
      pltpu.sync_copy(rows_v, psum_sp.at[dst_v.at[j]], add=True)
      if with_cnt:
        pltpu.sync_copy(ones_v, cnt_sp.at[dst_v.at[j]], add=True)
      return carry
    lax.fori_loop(0, CPW, step, 0)

    plsc.subcore_barrier()

    pltpu.sync_copy(psum_sp.at[pl.ds(base, RPT)],
                    out_sum.at[cid].at[pl.ds(base, RPT)])
    if with_cnt:
      pltpu.sync_copy(cnt_sp.at[pl.ds(base, RPT)],
                      out_cnt.at[cid].at[pl.ds(base, RPT)])

  return pl.kernel(body, mesh=mesh, out_type=tuple(out_type),
                   scratch_types=scratch_types)


_sc_segsum_cnt_call = _sc_segsum(True)
_sc_segsum_call = _sc_segsum(False)

BR = 1000


def _tc_layer(p, cnt2, xin, w_rel, w_root, b, relu):
  def tc_body(p_ref, c_ref, x_ref, wr_ref, wt_ref, b_ref, o_ref):
    s = p_ref[0] + p_ref[1]
    cnt = c_ref[0, :, 0:1] + c_ref[1, :, 0:1]
    agg = s * (1.0 / jnp.maximum(cnt, 1.0))
    y = (jnp.dot(agg, wr_ref[...], preferred_element_type=jnp.float32)
         + jnp.dot(x_ref[...], wt_ref[...], preferred_element_type=jnp.float32)
         + b_ref[...])
    o_ref[...] = jnp.maximum(y, 0.0) if relu else y

  return pl.pallas_call(
      tc_body,
      grid=(N // BR,),
      in_specs=[
          pl.BlockSpec((NC, BR, D), lambda i: (0, i, 0)),
          pl.BlockSpec((NC, BR, 16), lambda i: (0, i, 0)),
          pl.BlockSpec((BR, D), lambda i: (i, 0)),
          pl.BlockSpec((D, D), lambda i: (0, 0)),
          pl.BlockSpec((D, D), lambda i: (0, 0)),
          pl.BlockSpec((1, D), lambda i: (0, 0)),
      ],
      out_specs=pl.BlockSpec((BR, D), lambda i: (i, 0)),
      out_shape=jax.ShapeDtypeStruct((N, D), jnp.float32),
  )(p, cnt2, xin, w_rel, w_root, b)


@jax.jit
def kernel(x, edge_index, W1_rel, W1_root, b1, W2_rel, W2_root, b2):
  src = edge_index[0]
  dst = edge_index[1]
  pad = E_PAD - E
  src_p = jnp.concatenate(
      [src, jnp.zeros((pad,), jnp.int32)]).reshape(NW, CPW, C)
  dst_p = jnp.concatenate(
      [dst, jnp.full((pad,), N, jnp.int32)]).reshape(NW, CPW, C)

  psum1, cnt2 = _sc_segsum_cnt_call(x, src_p, dst_p)
  h = _tc_layer(psum1, cnt2, x, W1_rel, W1_root, b1.reshape(1, D), relu=True)
  psum2 = _sc_segsum_call(h, src_p, dst_p)
  out = _tc_layer(psum2, cnt2, h, W2_rel, W2_root, b2.reshape(1, D), relu=False)
  return out


# speedup vs baseline: 2.9204x; 2.9204x over previous
import functools

import jax
import jax.numpy as jnp
from jax import lax
from jax.experimental import pallas as pl
from jax.experimental.pallas import tpu as pltpu
from jax.experimental.pallas import tpu_sc as plsc

N = 10000
E = 320000
D = 128

NC = 2
NS = 16
NW = NC * NS
C = 128
CPW = 80
E_PAD = NW * CPW * C
N_PAD = 10112
RPT = N_PAD // NS

_MESH = plsc.VectorSubcoreMesh(core_axis_name="c", subcore_axis_name="s")


def _zero_fill(ref, val):
  rows, cols = ref.shape
  def body(i, carry):
    ref[i // (cols // 16), pl.ds((i % (cols // 16)) * 16, 16)] = (
        jnp.full((16,), val, jnp.float32))
    return carry
  lax.fori_loop(0, rows * (cols // 16), body, 0)


def _zero_spmem_slice(zsrc_v, sp, base):
  for k in range(RPT // C):
    pltpu.sync_copy(zsrc_v, sp.at[pl.ds(base + k * C, C)])
  rem = RPT % C
  if rem:
    pltpu.sync_copy(zsrc_v.at[pl.ds(0, rem)],
                    sp.at[pl.ds(base + (RPT // C) * C, rem)])


def _sc_segsum_body(table_hbm, src_hbm, dst_hbm, out_sum, src_v, dst_v,
                    rows_v, psum_sp, sem):
  cid = lax.axis_index("c")
  sid = lax.axis_index("s")
  wid = cid * NS + sid
  base = sid * RPT

  _zero_fill(rows_v, 0.0)
  _zero_spmem_slice(rows_v, psum_sp, base)

  plsc.subcore_barrier()

  ebase = wid * (CPW * C)

  def step(j, carry):
    off = ebase + j * C
    pltpu.sync_copy(src_hbm.at[pl.ds(off, C)], src_v)
    pltpu.sync_copy(dst_hbm.at[pl.ds(off, C)], dst_v)
    pltpu.async_copy(table_hbm.at[src_v], rows_v, sem).wait()
    pltpu.sync_copy(rows_v, psum_sp.at[dst_v], add=True)
    return carry
  lax.fori_loop(0, CPW, step, 0)

  plsc.subcore_barrier()

  pltpu.sync_copy(psum_sp.at[pl.ds(base, RPT)],
                  out_sum.at[cid].at[pl.ds(base, RPT)])


_sc_segsum_call = pl.kernel(
    _sc_segsum_body, mesh=_MESH,
    out_type=jax.ShapeDtypeStruct((NC, N_PAD, D), jnp.float32),
    scratch_types=[
        pltpu.VMEM((C,), jnp.int32),
        pltpu.VMEM((C,), jnp.int32),
        pltpu.VMEM((C, D), jnp.float32),
        pltpu.VMEM_SHARED((N_PAD, D), jnp.float32),
        pltpu.SemaphoreType.DMA,
    ])


def _sc_count_body(dst_hbm, out_cnt, dst_v, ones_v, cnt_sp, sem):
  cid = lax.axis_index("c")
  sid = lax.axis_index("s")
  wid = cid * NS + sid
  base = sid * RPT

  _zero_fill(ones_v, 0.0)
  _zero_spmem_slice(ones_v, cnt_sp, base)
  _zero_fill(ones_v, 1.0)

  plsc.subcore_barrier()

  ebase = wid * (CPW * C)

  def step(j, carry):
    off = ebase + j * C
    pltpu.sync_copy(dst_hbm.at[pl.ds(off, C)], dst_v)
    pltpu.sync_copy(ones_v, cnt_sp.at[dst_v], add=True)
    return carry
  lax.fori_loop(0, CPW, step, 0)

  plsc.subcore_barrier()

  pltpu.sync_copy(cnt_sp.at[pl.ds(base, RPT)],
                  out_cnt.at[cid].at[pl.ds(base, RPT)])


_sc_count_call = pl.kernel(
    _sc_count_body, mesh=_MESH,
    out_type=jax.ShapeDtypeStruct((NC, N_PAD, D), jnp.float32),
    scratch_types=[
        pltpu.VMEM((C,), jnp.int32),
        pltpu.VMEM((C, D), jnp.float32),
        pltpu.VMEM_SHARED((N_PAD, D), jnp.float32),
        pltpu.SemaphoreType.DMA,
    ])

BR = 1000


def _tc_layer(p, cnt2, xin, w_rel, w_root, b, relu):
  def tc_body(p_ref, c_ref, x_ref, wr_ref, wt_ref, b_ref, o_ref):
    s = p_ref[0] + p_ref[1]
    cnt = c_ref[0, :, 0:1] + c_ref[1, :, 0:1]
    agg = s * (1.0 / jnp.maximum(cnt, 1.0))
    y = (jnp.dot(agg, wr_ref[...], preferred_element_type=jnp.float32)
         + jnp.dot(x_ref[...], wt_ref[...], preferred_element_type=jnp.float32)
         + b_ref[...])
    o_ref[...] = jnp.maximum(y, 0.0) if relu else y

  return pl.pallas_call(
      tc_body,
      grid=(N // BR,),
      in_specs=[
          pl.BlockSpec((NC, BR, D), lambda i: (0, i, 0)),
          pl.BlockSpec((NC, BR, D), lambda i: (0, i, 0)),
          pl.BlockSpec((BR, D), lambda i: (i, 0)),
          pl.BlockSpec((D, D), lambda i: (0, 0)),
          pl.BlockSpec((D, D), lambda i: (0, 0)),
          pl.BlockSpec((1, D), lambda i: (0, 0)),
      ],
      out_specs=pl.BlockSpec((BR, D), lambda i: (i, 0)),
      out_shape=jax.ShapeDtypeStruct((N, D), jnp.float32),
  )(p, cnt2, xin, w_rel, w_root, b)


@jax.jit
def kernel(x, edge_index, W1_rel, W1_root, b1, W2_rel, W2_root, b2):
  src = edge_index[0]
  dst = edge_index[1]
  pad = E_PAD - E
  src_p = jnp.concatenate([src, jnp.zeros((pad,), jnp.int32)])
  dst_p = jnp.concatenate([dst, jnp.full((pad,), N, jnp.int32)])

  cnt2 = _sc_count_call(dst_p)
  psum1 = _sc_segsum_call(x, src_p, dst_p)
  h = _tc_layer(psum1, cnt2, x, W1_rel, W1_root, b1.reshape(1, D), relu=True)
  psum2 = _sc_segsum_call(h, src_p, dst_p)
  out = _tc_layer(psum2, cnt2, h, W2_rel, W2_root, b2.reshape(1, D), relu=False)
  return out


# speedup vs baseline: 3.5201x; 1.2054x over previous
import functools

import jax
import jax.numpy as jnp
from jax import lax
from jax.experimental import pallas as pl
from jax.experimental.pallas import tpu as pltpu
from jax.experimental.pallas import tpu_sc as plsc

N = 10000
E = 320000
D = 128

NC = 2
NS = 16
NW = NC * NS
C = 128
CPW = 80
E_PAD = NW * CPW * C
N_PAD = 10112
RPT = N_PAD // NS

_MESH = plsc.VectorSubcoreMesh(core_axis_name="c", subcore_axis_name="s")


def _zero_fill(ref, val):
  rows, cols = ref.shape
  def body(i, carry):
    ref[i // (cols // 16), pl.ds((i % (cols // 16)) * 16, 16)] = (
        jnp.full((16,), val, jnp.float32))
    return carry
  lax.fori_loop(0, rows * (cols // 16), body, 0)


def _zero_spmem_slice(zsrc_v, sp, base):
  for k in range(RPT // C):
    pltpu.sync_copy(zsrc_v, sp.at[pl.ds(base + k * C, C)])
  rem = RPT % C
  if rem:
    pltpu.sync_copy(zsrc_v.at[pl.ds(0, rem)],
                    sp.at[pl.ds(base + (RPT // C) * C, rem)])


def _sc_segsum_body(table_hbm, src_hbm, dst_hbm, out_sum,
                    src_a, dst_a, src_b, dst_b, rows_a, rows_b,
                    psum_sp, gs_a, gs_b, ss_a, ss_b):
  cid = lax.axis_index("c")
  sid = lax.axis_index("s")
  wid = cid * NS + sid
  base = sid * RPT

  _zero_fill(rows_a, 0.0)
  _zero_spmem_slice(rows_a, psum_sp, base)

  plsc.subcore_barrier()

  ebase = wid * (CPW * C)

  def stage(idx_ref, which, off):
    pltpu.sync_copy(which.at[pl.ds(off, C)], idx_ref)

  stage(src_a, src_hbm, ebase)
  stage(dst_a, dst_hbm, ebase)
  pltpu.async_copy(table_hbm.at[src_a], rows_a, gs_a)

  def pair(g, carry):
    off_b = ebase + (2 * g + 1) * C
    @pl.when(g > 0)
    def _():
      pltpu.make_async_copy(rows_b, psum_sp.at[dst_b], ss_b).wait()
    stage(src_b, src_hbm, off_b)
    stage(dst_b, dst_hbm, off_b)
    pltpu.async_copy(table_hbm.at[src_b], rows_b, gs_b)
    pltpu.make_async_copy(table_hbm.at[src_a], rows_a, gs_a).wait()
    pltpu.async_copy(rows_a, psum_sp.at[dst_a], ss_a, add=True)
    pltpu.make_async_copy(rows_a, psum_sp.at[dst_a], ss_a).wait()
    @pl.when(g < CPW // 2 - 1)
    def _():
      off_a = ebase + (2 * g + 2) * C
      stage(src_a, src_hbm, off_a)
      stage(dst_a, dst_hbm, off_a)
      pltpu.async_copy(table_hbm.at[src_a], rows_a, gs_a)
    pltpu.make_async_copy(table_hbm.at[src_b], rows_b, gs_b).wait()
    pltpu.async_copy(rows_b, psum_sp.at[dst_b], ss_b, add=True)
    return carry
  lax.fori_loop(0, CPW // 2, pair, 0)

  pltpu.make_async_copy(rows_b, psum_sp.at[dst_b], ss_b).wait()

  plsc.subcore_barrier()

  pltpu.sync_copy(psum_sp.at[pl.ds(base, RPT)],
                  out_sum.at[cid].at[pl.ds(base, RPT)])


_sc_segsum_call = pl.kernel(
    _sc_segsum_body, mesh=_MESH,
    out_type=jax.ShapeDtypeStruct((NC, N_PAD, D), jnp.float32),
    scratch_types=[
        pltpu.VMEM((C,), jnp.int32),
        pltpu.VMEM((C,), jnp.int32),
        pltpu.VMEM((C,), jnp.int32),
        pltpu.VMEM((C,), jnp.int32),
        pltpu.VMEM((C, D), jnp.float32),
        pltpu.VMEM((C, D), jnp.float32),
        pltpu.VMEM_SHARED((N_PAD, D), jnp.float32),
        pltpu.SemaphoreType.DMA,
        pltpu.SemaphoreType.DMA,
        pltpu.SemaphoreType.DMA,
        pltpu.SemaphoreType.DMA,
    ])


def _sc_count_body(dst_hbm, out_cnt, dst_v, ones_v, cnt_sp, sem):
  cid = lax.axis_index("c")
  sid = lax.axis_index("s")
  wid = cid * NS + sid
  base = sid * RPT

  _zero_fill(ones_v, 0.0)
  _zero_spmem_slice(ones_v, cnt_sp, base)
  _zero_fill(ones_v, 1.0)

  plsc.subcore_barrier()

  ebase = wid * (CPW * C)

  def step(j, carry):
    off = ebase + j * C
    pltpu.sync_copy(dst_hbm.at[pl.ds(off, C)], dst_v)
    pltpu.sync_copy(ones_v, cnt_sp.at[dst_v], add=True)
    return carry
  lax.fori_loop(0, CPW, step, 0)

  plsc.subcore_barrier()

  pltpu.sync_copy(cnt_sp.at[pl.ds(base, RPT)],
                  out_cnt.at[cid].at[pl.ds(base, RPT)])


_sc_count_call = pl.kernel(
    _sc_count_body, mesh=_MESH,
    out_type=jax.ShapeDtypeStruct((NC, N_PAD, D), jnp.float32),
    scratch_types=[
        pltpu.VMEM((C,), jnp.int32),
        pltpu.VMEM((C, D), jnp.float32),
        pltpu.VMEM_SHARED((N_PAD, D), jnp.float32),
        pltpu.SemaphoreType.DMA,
    ])

BR = 1000


def _tc_layer(p, cnt2, xin, w_rel, w_root, b, relu):
  def tc_body(p_ref, c_ref, x_ref, wr_ref, wt_ref, b_ref, o_ref):
    s = p_ref[0] + p_ref[1]
    cnt = c_ref[0, :, 0:1] + c_ref[1, :, 0:1]
    agg = s * (1.0 / jnp.maximum(cnt, 1.0))
    y = (jnp.dot(agg, wr_ref[...], preferred_element_type=jnp.float32)
         + jnp.dot(x_ref[...], wt_ref[...], preferred_element_type=jnp.float32)
         + b_ref[...])
    o_ref[...] = jnp.maximum(y, 0.0) if relu else y

  return pl.pallas_call(
      tc_body,
      grid=(N // BR,),
      in_specs=[
          pl.BlockSpec((NC, BR, D), lambda i: (0, i, 0)),
          pl.BlockSpec((NC, BR, D), lambda i: (0, i, 0)),
          pl.BlockSpec((BR, D), lambda i: (i, 0)),
          pl.BlockSpec((D, D), lambda i: (0, 0)),
          pl.BlockSpec((D, D), lambda i: (0, 0)),
          pl.BlockSpec((1, D), lambda i: (0, 0)),
      ],
      out_specs=pl.BlockSpec((BR, D), lambda i: (i, 0)),
      out_shape=jax.ShapeDtypeStruct((N, D), jnp.float32),
  )(p, cnt2, xin, w_rel, w_root, b)


@jax.jit
def kernel(x, edge_index, W1_rel, W1_root, b1, W2_rel, W2_root, b2):
  src = edge_index[0]
  dst = edge_index[1]
  pad = E_PAD - E
  src_p = jnp.concatenate([src, jnp.zeros((pad,), jnp.int32)])
  dst_p = jnp.concatenate([dst, jnp.full((pad,), N, jnp.int32)])

  cnt2 = _sc_count_call(dst_p)
  psum1 = _sc_segsum_call(x, src_p, dst_p)
  h = _tc_layer(psum1, cnt2, x, W1_rel, W1_root, b1.reshape(1, D), relu=True)
  psum2 = _sc_segsum_call(h, src_p, dst_p)
  out = _tc_layer(psum2, cnt2, h, W2_rel, W2_root, b2.reshape(1, D), relu=False)
  return out
